# baseline (speedup 1.0000x reference)
import functools

import jax
import jax.numpy as jnp
from jax.experimental import pallas as pl
from jax.experimental.pallas import tpu as pltpu

_BN_EPS = 1e-5


def _round_up(v, m):
    return (v + m - 1) // m * m


def _col_index(GP, W):
    if GP % W == 0:
        it = jax.lax.broadcasted_iota(jnp.int32, (GP // W, W, 1), 1)
        return it.reshape(GP, 1)
    return jax.lax.broadcasted_iota(jnp.int32, (GP, 1), 0) % W


def _conv_stats_kernel(x_ref, w_ref, y_ref, ps_ref, ext_ref, c3_ref,
                       *, B, G, W, C, Cout, K, MARG, GP):
    p = K // 2
    col = _col_index(GP, W)
    base = MARG - p * W - p
    for b in range(B):
        ext_ref[b, 0:MARG, :] = jnp.zeros((MARG, C), jnp.bfloat16)
        ext_ref[b, MARG + G:, :] = jnp.zeros_like(ext_ref[b, MARG + G:, :])
        ext_ref[b, MARG:MARG + G, :] = jnp.transpose(
            x_ref[b].reshape(C, G), (1, 0)).astype(jnp.bfloat16)

        for kw in range(K):
            src = ext_ref[b, base + kw:base + kw + GP, :]
            if kw < p:
                src = jnp.where(col >= p - kw, src, 0)
            elif kw > p:
                src = jnp.where(col < W - (kw - p), src, 0)
            c3_ref[b, :, kw * C:(kw + 1) * C] = src

        pm = jnp.dot(c3_ref[b], w_ref[...],
                     preferred_element_type=jnp.float32)

        acc = pm[0:G, 0:Cout]
        for kh in range(1, K):
            acc = acc + pm[kh * W:kh * W + G, kh * Cout:(kh + 1) * Cout]

        ps_ref[b, 0, :] = jnp.sum(acc, axis=0)
        ps_ref[b, 1, :] = jnp.sum(acc * acc, axis=0)

        y_ref[b] = jnp.transpose(acc, (1, 0)).astype(y_ref.dtype)


def _bn_apply_kernel(ps_ref, gb_ref, y_ref, o_ref, *, B, m_true, HW):
    inv_m = 1.0 / m_true
    mean = jnp.sum(ps_ref[:, 0, :], axis=0, keepdims=True) * inv_m
    ex2 = jnp.sum(ps_ref[:, 1, :], axis=0, keepdims=True) * inv_m
    var = ex2 - mean * mean
    inv_std = jax.lax.rsqrt(var + _BN_EPS)
    scale = inv_std * gb_ref[0:1, :]
    shift = gb_ref[1:2, :] - mean * scale
    scale_c = jnp.transpose(scale, (1, 0))
    shift_c = jnp.transpose(shift, (1, 0))
    C = y_ref.shape[1]
    for b in range(B):
        y = y_ref[b].astype(jnp.float32)
        o_ref[b] = jnp.maximum(y * scale_c + shift_c, 0.0).reshape(C, *HW)


def _conv_bn_relu(x_nchw, weight, gamma, beta):
    N, Cin, H, W = x_nchw.shape
    Cout, _, K, _ = weight.shape
    G = H * W
    p = K // 2
    MARG = _round_up(p * W + p, 8)
    GP = _round_up(G + 2 * p * W, 8)
    EXT = MARG + G + MARG
    m_true = float(N * G)
    B = 4 if N % 4 == 0 else (2 if N % 2 == 0 else 1)


    wf = jnp.concatenate(
        [jnp.transpose(weight[:, :, kh, :], (2, 1, 0)).reshape(K * Cin, Cout)
         for kh in range(K)], axis=1)
    wf = wf.astype(jnp.bfloat16)

    gb = jnp.stack([gamma, beta], axis=0).astype(jnp.float32)

    y, ps = pl.pallas_call(
        functools.partial(_conv_stats_kernel, B=B, G=G, W=W, C=Cin,
                          Cout=Cout, K=K, MARG=MARG, GP=GP),
        out_shape=(jax.ShapeDtypeStruct((N, Cout, G), jnp.bfloat16),
                   jax.ShapeDtypeStruct((N, 2, Cout), jnp.float32)),
        grid=(N // B,),
        in_specs=[pl.BlockSpec((B, Cin, H, W), lambda i: (i, 0, 0, 0)),
                  pl.BlockSpec((K * Cin, K * Cout), lambda i: (0, 0))],
        out_specs=[pl.BlockSpec((B, Cout, G), lambda i: (i, 0, 0)),
                   pl.BlockSpec((B, 2, Cout), lambda i: (i, 0, 0))],
        scratch_shapes=[pltpu.VMEM((B, EXT, Cin), jnp.bfloat16),
                        pltpu.VMEM((B, GP, K * Cin), jnp.bfloat16)],
        compiler_params=pltpu.CompilerParams(
            dimension_semantics=("parallel",)),
    )(x_nchw, wf)

    out = pl.pallas_call(
        functools.partial(_bn_apply_kernel, B=B, m_true=m_true, HW=(H, W)),
        out_shape=jax.ShapeDtypeStruct((N, Cout, H, W), jnp.float32),
        grid=(N // B,),
        in_specs=[pl.BlockSpec((N, 2, Cout), lambda i: (0, 0, 0)),
                  pl.BlockSpec((2, Cout), lambda i: (0, 0)),
                  pl.BlockSpec((B, Cout, G), lambda i: (i, 0, 0))],
        out_specs=pl.BlockSpec((B, Cout, H, W), lambda i: (i, 0, 0, 0)),
        compiler_params=pltpu.CompilerParams(
            dimension_semantics=("parallel",)),
    )(ps, gb, y)

    return out


def kernel(x, weight, bias, gamma, beta):
    del bias
    return _conv_bn_relu(x, weight, gamma, beta)


# speedup vs baseline: 1.2509x; 1.2509x over previous
import functools

import jax
import jax.numpy as jnp
from jax.experimental import pallas as pl
from jax.experimental.pallas import tpu as pltpu

_BN_EPS = 1e-5


def _round_up(v, m):
    return (v + m - 1) // m * m


def _col_index(GP, W):
    if GP % W == 0:
        it = jax.lax.broadcasted_iota(jnp.int32, (GP // W, W, 1), 1)
        return it.reshape(GP, 1)
    return jax.lax.broadcasted_iota(jnp.int32, (GP, 1), 0) % W


def _conv_stats_kernel(x_ref, w_ref, y_ref, ps_ref, ext_ref, c3_ref,
                       *, B, G, W, C, Cout, K, MARG, GP):
    p = K // 2
    col = _col_index(GP, W)
    base = MARG - p * W - p
    for b in range(B):
        ext_ref[b, 0:MARG, :] = jnp.zeros((MARG, C), jnp.bfloat16)
        ext_ref[b, MARG + G:, :] = jnp.zeros_like(ext_ref[b, MARG + G:, :])
        ext_ref[b, MARG:MARG + G, :] = jnp.transpose(x_ref[b], (1, 0))

        for kw in range(K):
            src = ext_ref[b, base + kw:base + kw + GP, :]
            if kw < p:
                src = jnp.where(col >= p - kw, src, 0)
            elif kw > p:
                src = jnp.where(col < W - (kw - p), src, 0)
            c3_ref[b, :, kw * C:(kw + 1) * C] = src

        pm = jnp.dot(c3_ref[b], w_ref[...],
                     preferred_element_type=jnp.float32)

        acc = pm[0:G, 0:Cout]
        for kh in range(1, K):
            acc = acc + pm[kh * W:kh * W + G, kh * Cout:(kh + 1) * Cout]

        ps_ref[b, 0, :] = jnp.sum(acc, axis=0)
        ps_ref[b, 1, :] = jnp.sum(acc * acc, axis=0)

        y_ref[b] = jnp.transpose(acc, (1, 0)).astype(y_ref.dtype)


def _bn_apply_kernel(ps_ref, gb_ref, y_ref, o_ref, *, B, m_true):
    inv_m = 1.0 / m_true
    mean = jnp.sum(ps_ref[:, 0, :], axis=0, keepdims=True) * inv_m
    ex2 = jnp.sum(ps_ref[:, 1, :], axis=0, keepdims=True) * inv_m
    var = ex2 - mean * mean
    inv_std = jax.lax.rsqrt(var + _BN_EPS)
    scale = inv_std * gb_ref[0:1, :]
    shift = gb_ref[1:2, :] - mean * scale
    scale_c = jnp.transpose(scale, (1, 0))
    shift_c = jnp.transpose(shift, (1, 0))
    for b in range(B):
        y = y_ref[b].astype(jnp.float32)
        o_ref[b] = jnp.maximum(y * scale_c + shift_c, 0.0)


def _conv_bn_relu(x_nchw, weight, gamma, beta):
    N, Cin, H, W = x_nchw.shape
    Cout, _, K, _ = weight.shape
    G = H * W
    p = K // 2
    MARG = _round_up(p * W + p, 8)
    GP = _round_up(G + 2 * p * W, 8)
    EXT = MARG + G + MARG
    m_true = float(N * G)
    B = 4 if N % 4 == 0 else (2 if N % 2 == 0 else 1)

    x3 = x_nchw.reshape(N, Cin, G).astype(jnp.bfloat16)

    wf = jnp.concatenate(
        [jnp.transpose(weight[:, :, kh, :], (2, 1, 0)).reshape(K * Cin, Cout)
         for kh in range(K)], axis=1)
    wf = wf.astype(jnp.bfloat16)

    gb = jnp.stack([gamma, beta], axis=0).astype(jnp.float32)

    y, ps = pl.pallas_call(
        functools.partial(_conv_stats_kernel, B=B, G=G, W=W, C=Cin,
                          Cout=Cout, K=K, MARG=MARG, GP=GP),
        out_shape=(jax.ShapeDtypeStruct((N, Cout, G), jnp.bfloat16),
                   jax.ShapeDtypeStruct((N, 2, Cout), jnp.float32)),
        grid=(N // B,),
        in_specs=[pl.BlockSpec((B, Cin, G), lambda i: (i, 0, 0)),
                  pl.BlockSpec((K * Cin, K * Cout), lambda i: (0, 0))],
        out_specs=[pl.BlockSpec((B, Cout, G), lambda i: (i, 0, 0)),
                   pl.BlockSpec((B, 2, Cout), lambda i: (i, 0, 0))],
        scratch_shapes=[pltpu.VMEM((B, EXT, Cin), jnp.bfloat16),
                        pltpu.VMEM((B, GP, K * Cin), jnp.bfloat16)],
        compiler_params=pltpu.CompilerParams(
            dimension_semantics=("parallel",)),
    )(x3, wf)

    out = pl.pallas_call(
        functools.partial(_bn_apply_kernel, B=B, m_true=m_true),
        out_shape=jax.ShapeDtypeStruct((N, Cout, G), jnp.float32),
        grid=(N // B,),
        in_specs=[pl.BlockSpec((N, 2, Cout), lambda i: (0, 0, 0)),
                  pl.BlockSpec((2, Cout), lambda i: (0, 0)),
                  pl.BlockSpec((B, Cout, G), lambda i: (i, 0, 0))],
        out_specs=pl.BlockSpec((B, Cout, G), lambda i: (i, 0, 0)),
        compiler_params=pltpu.CompilerParams(
            dimension_semantics=("parallel",)),
    )(ps, gb, y)

    return out.reshape(N, Cout, H, W)


def kernel(x, weight, bias, gamma, beta):
    del bias
    return _conv_bn_relu(x, weight, gamma, beta)


# speedup vs baseline: 1.3680x; 1.0937x over previous
import functools

import jax
import jax.numpy as jnp
from jax.experimental import pallas as pl
from jax.experimental.pallas import tpu as pltpu

_BN_EPS = 1e-5


def _round_up(v, m):
    return (v + m - 1) // m * m


def _col_index(GP, W):
    if GP % W == 0:
        it = jax.lax.broadcasted_iota(jnp.int32, (GP // W, W, 1), 1)
        return it.reshape(GP, 1)
    return jax.lax.broadcasted_iota(jnp.int32, (GP, 1), 0) % W


def _conv_stats_kernel(x_ref, w_ref, y_ref, ps_ref, ext_ref, c3_ref,
                       *, B, G, W, C, Cout, K, MARG, GP):
    p = K // 2
    col = _col_index(GP, W)
    base = MARG - p * W - p
    for b in range(B):
        ext_ref[b, 0:MARG, :] = jnp.zeros((MARG, C), jnp.bfloat16)
        ext_ref[b, MARG + G:, :] = jnp.zeros_like(ext_ref[b, MARG + G:, :])
        ext_ref[b, MARG:MARG + G, :] = jnp.transpose(
            x_ref[b], (1, 0)).astype(jnp.bfloat16)

        for kw in range(K):
            src = ext_ref[b, base + kw:base + kw + GP, :]
            if kw < p:
                src = jnp.where(col >= p - kw, src, 0)
            elif kw > p:
                src = jnp.where(col < W - (kw - p), src, 0)
            c3_ref[b, :, kw * C:(kw + 1) * C] = src

        pm = jnp.dot(c3_ref[b], w_ref[...],
                     preferred_element_type=jnp.float32)

        acc = pm[0:G, 0:Cout]
        for kh in range(1, K):
            acc = acc + pm[kh * W:kh * W + G, kh * Cout:(kh + 1) * Cout]

        ps_ref[b, 0, :] = jnp.sum(acc, axis=0)
        ps_ref[b, 1, :] = jnp.sum(acc * acc, axis=0)

        y_ref[b] = jnp.transpose(acc, (1, 0)).astype(y_ref.dtype)


def _bn_apply_kernel(ps_ref, gb_ref, y_ref, o_ref, *, B, m_true):
    inv_m = 1.0 / m_true
    mean = jnp.sum(ps_ref[:, 0, :], axis=0, keepdims=True) * inv_m
    ex2 = jnp.sum(ps_ref[:, 1, :], axis=0, keepdims=True) * inv_m
    var = ex2 - mean * mean
    inv_std = jax.lax.rsqrt(var + _BN_EPS)
    scale = inv_std * gb_ref[0:1, :]
    shift = gb_ref[1:2, :] - mean * scale
    scale_c = jnp.transpose(scale, (1, 0))
    shift_c = jnp.transpose(shift, (1, 0))
    for b in range(B):
        y = y_ref[b].astype(jnp.float32)
        o_ref[b] = jnp.maximum(y * scale_c + shift_c, 0.0)


def _conv_bn_relu(x_nchw, weight, gamma, beta):
    N, Cin, H, W = x_nchw.shape
    Cout, _, K, _ = weight.shape
    G = H * W
    p = K // 2
    MARG = _round_up(p * W + p, 8)
    GP = _round_up(G + 2 * p * W, 8)
    EXT = MARG + G + MARG
    m_true = float(N * G)
    B = next(b for b in (8, 4, 2, 1) if N % b == 0)

    x3 = x_nchw.reshape(N, Cin, G)

    wf = jnp.concatenate(
        [jnp.transpose(weight[:, :, kh, :], (2, 1, 0)).reshape(K * Cin, Cout)
         for kh in range(K)], axis=1)
    wf = wf.astype(jnp.bfloat16)

    gb = jnp.stack([gamma, beta], axis=0).astype(jnp.float32)

    y, ps = pl.pallas_call(
        functools.partial(_conv_stats_kernel, B=B, G=G, W=W, C=Cin,
                          Cout=Cout, K=K, MARG=MARG, GP=GP),
        out_shape=(jax.ShapeDtypeStruct((N, Cout, G), jnp.bfloat16),
                   jax.ShapeDtypeStruct((N, 2, Cout), jnp.float32)),
        grid=(N // B,),
        in_specs=[pl.BlockSpec((B, Cin, G), lambda i: (i, 0, 0)),
                  pl.BlockSpec((K * Cin, K * Cout), lambda i: (0, 0))],
        out_specs=[pl.BlockSpec((B, Cout, G), lambda i: (i, 0, 0)),
                   pl.BlockSpec((B, 2, Cout), lambda i: (i, 0, 0))],
        scratch_shapes=[pltpu.VMEM((B, EXT, Cin), jnp.bfloat16),
                        pltpu.VMEM((B, GP, K * Cin), jnp.bfloat16)],
        compiler_params=pltpu.CompilerParams(
            dimension_semantics=("parallel",)),
    )(x3, wf)

    out = pl.pallas_call(
        functools.partial(_bn_apply_kernel, B=B, m_true=m_true),
        out_shape=jax.ShapeDtypeStruct((N, Cout, G), jnp.float32),
        grid=(N // B,),
        in_specs=[pl.BlockSpec((N, 2, Cout), lambda i: (0, 0, 0)),
                  pl.BlockSpec((2, Cout), lambda i: (0, 0)),
                  pl.BlockSpec((B, Cout, G), lambda i: (i, 0, 0))],
        out_specs=pl.BlockSpec((B, Cout, G), lambda i: (i, 0, 0)),
        compiler_params=pltpu.CompilerParams(
            dimension_semantics=("parallel",)),
    )(ps, gb, y)

    return out.reshape(N, Cout, H, W)


def kernel(x, weight, bias, gamma, beta):
    del bias
    return _conv_bn_relu(x, weight, gamma, beta)
